# baseline (speedup 1.0000x reference)
import functools

import jax
import jax.numpy as jnp
from jax import lax
from jax.experimental import pallas as pl
from jax.experimental.pallas import tpu as pltpu
from jax.experimental.pallas import tpu_sc as plsc


def _make_sc_fill(N, K):
    info = plsc.get_sparse_core_info()
    NC, NS, L = info.num_cores, info.num_subcores, info.num_lanes
    CH = N // (NC * NS)
    CB = min(4096, CH)
    mesh = plsc.VectorSubcoreMesh(core_axis_name="c", subcore_axis_name="s")

    @functools.partial(
        pl.kernel,
        mesh=mesh,
        out_type=jax.ShapeDtypeStruct((2, K, N), jnp.float32),
        scratch_types=[
            pltpu.VMEM((K,), jnp.float32),
            pltpu.VMEM((K, CB), jnp.float32),
            pltpu.SemaphoreType.DMA,
        ],
    )
    def sc_fill(r0_hbm, out_hbm, r0_v, stage, sem):
        wid = lax.axis_index("s") * NC + lax.axis_index("c")
        pltpu.sync_copy(r0_hbm, r0_v)
        r0_vec = r0_v[...]
        for k in range(K):
            vec = jnp.full((L,), r0_vec[k], dtype=jnp.float32)

            def body(j, _, k=k, vec=vec):
                stage[k, pl.ds(j * L, L)] = vec
                return 0

            lax.fori_loop(0, CB // L, body, 0)
        base = wid * CH
        copies = [
            pltpu.async_copy(
                stage, out_hbm.at[0, :, pl.ds(base + t * CB, CB)], sem
            )
            for t in range(CH // CB)
        ]
        for cp in copies:
            cp.wait()

    return sc_fill


def _tc_body(x_ref, w_ref, b_ref, buf_ref, out_ref):
    del buf_ref
    out_ref[0] = (
        jnp.dot(w_ref[...], x_ref[0], preferred_element_type=jnp.float32)
        + b_ref[...]
    )


def kernel(x, token, W, b):
    B, N, C = x.shape
    K = W.shape[0]
    b_col = b.reshape(K, 1)
    r0_col = W @ token.reshape(C, 1) + b_col
    xt = jnp.transpose(x, (0, 2, 1))

    buf = _make_sc_fill(N, K)(r0_col.reshape(K))

    Bn = min(131072, N)
    grid = (N // Bn,)
    out_t = pl.pallas_call(
        _tc_body,
        grid=grid,
        in_specs=[
            pl.BlockSpec((1, C, Bn), lambda i: (1, 0, i)),
            pl.BlockSpec((K, C), lambda i: (0, 0)),
            pl.BlockSpec((K, 1), lambda i: (0, 0)),
            pl.BlockSpec(memory_space=pl.ANY),
        ],
        out_specs=pl.BlockSpec((1, K, Bn), lambda i: (1, 0, i)),
        out_shape=jax.ShapeDtypeStruct((B, K, N), x.dtype),
        input_output_aliases={3: 0},
    )(xt, W, b_col, buf)
    return jnp.transpose(out_t, (0, 2, 1))


# speedup vs baseline: 1.0116x; 1.0116x over previous
import functools

import jax
import jax.numpy as jnp
from jax import lax
from jax.experimental import pallas as pl
from jax.experimental.pallas import tpu as pltpu
from jax.experimental.pallas import tpu_sc as plsc


def _make_sc_fill(N, K):
    info = plsc.get_sparse_core_info()
    NC, NS, L = info.num_cores, info.num_subcores, info.num_lanes
    CH = N // (NC * NS)
    CB = min(4096, CH)
    mesh = plsc.VectorSubcoreMesh(core_axis_name="c", subcore_axis_name="s")

    @functools.partial(
        pl.kernel,
        mesh=mesh,
        out_type=jax.ShapeDtypeStruct((2, K, N), jnp.float32),
        scratch_types=[
            pltpu.VMEM((K,), jnp.float32),
            pltpu.VMEM((K, CB), jnp.float32),
            pltpu.SemaphoreType.DMA,
        ],
    )
    def sc_fill(r0_hbm, out_hbm, r0_v, stage, sem):
        wid = lax.axis_index("s") * NC + lax.axis_index("c")
        pltpu.sync_copy(r0_hbm, r0_v)
        r0_vec = r0_v[...]
        for k in range(K):
            vec = jnp.full((L,), r0_vec[k], dtype=jnp.float32)

            def body(j, _, k=k, vec=vec):
                stage[k, pl.ds(j * L, L)] = vec
                return 0

            lax.fori_loop(0, CB // L, body, 0)
        base = wid * CH
        copies = [
            pltpu.async_copy(
                stage, out_hbm.at[0, :, pl.ds(base + t * CB, CB)], sem
            )
            for t in range(CH // CB)
        ]
        for cp in copies:
            cp.wait()

    return sc_fill


def _tc_body(x_ref, w_ref, b_ref, buf_ref, out_ref):
    del buf_ref
    out_ref[0] = (
        jnp.dot(w_ref[...], x_ref[0], preferred_element_type=jnp.float32)
        + b_ref[...]
    )


def kernel(x, token, W, b):
    B, N, C = x.shape
    K = W.shape[0]
    b_col = b.reshape(K, 1)
    r0_col = W @ token.reshape(C, 1) + b_col
    xt = jnp.transpose(x, (0, 2, 1))

    buf = _make_sc_fill(N, K)(r0_col.reshape(K))

    Bn = min(262144, N)
    grid = (N // Bn,)
    out_t = pl.pallas_call(
        _tc_body,
        grid=grid,
        in_specs=[
            pl.BlockSpec((1, C, Bn), lambda i: (1, 0, i)),
            pl.BlockSpec((K, C), lambda i: (0, 0)),
            pl.BlockSpec((K, 1), lambda i: (0, 0)),
            pl.BlockSpec(memory_space=pl.ANY),
        ],
        out_specs=pl.BlockSpec((1, K, Bn), lambda i: (1, 0, i)),
        out_shape=jax.ShapeDtypeStruct((B, K, N), x.dtype),
        input_output_aliases={3: 0},
    )(xt, W, b_col, buf)
    return jnp.transpose(out_t, (0, 2, 1))
